# baseline (speedup 1.0000x reference)
import functools

import numpy as np
import jax
import jax.numpy as jnp
from jax import lax
from jax.experimental import pallas as pl
from jax.experimental.pallas import tpu as pltpu
from jax.experimental.pallas import tpu_sc as plsc

_BATCH = 4
_MAX_LEN = 2048
_D = 1024

_NC = 2
_NS = 16
_NW = _NC * _NS
_L = 16

_P_W = _MAX_LEN // _NW
_PER_W = _BATCH * _P_W
_CH = 16
_NQ = _P_W // _CH
_NCH = _BATCH * _NQ
_NR = 3


def _positional_encoding() -> np.ndarray:
    pos = np.arange(_MAX_LEN, dtype=np.float32)[:, None]
    i = np.arange(0, _D, 2, dtype=np.float32)
    div = np.exp(-np.log(10000.0) * i / _D)
    pe = np.zeros((_MAX_LEN, _D), dtype=np.float32)
    pe[:, 0::2] = np.sin(pos * div)
    pe[:, 1::2] = np.cos(pos * div)
    return pe


def _pack_pos_bf16() -> np.ndarray:
    pe = _positional_encoding()
    u = pe.view(np.uint32)
    bf = ((u + 0x7FFF + ((u >> 16) & 1)) >> 16).astype(np.uint32)
    bf = bf.reshape(_MAX_LEN, _D // 32, 2, 16)
    packed = bf[:, :, 0, :] | (bf[:, :, 1, :] << 16)
    return packed.reshape(_MAX_LEN, _D // 2).astype(np.uint32).view(np.int32)


_POS_PACKED = _pack_pos_bf16()
_MASK_HI = np.int32(-65536)

_mesh = plsc.VectorSubcoreMesh(core_axis_name="c", subcore_axis_name="s")


@functools.partial(
    pl.kernel,
    mesh=_mesh,
    out_type=jax.ShapeDtypeStruct((_BATCH * _MAX_LEN, _D), jnp.float32),
    scratch_types=(
        [pltpu.VMEM((_PER_W,), jnp.int32),
         pltpu.VMEM((_P_W, _D // 2), jnp.int32)]
        + [pltpu.VMEM((_CH, _D), jnp.float32)] * _NR
        + [pltpu.SemaphoreType.DMA] * (1 + 2 * _NR)
    ),
)
def _emb_kernel(x_hbm, pos_hbm, table_hbm, out_hbm, idx_v, pos_v,
                rows0, rows1, rows2,
                hsem, gsem0, gsem1, gsem2, osem0, osem1, osem2):
    rows = (rows0, rows1, rows2)
    gsem = (gsem0, gsem1, gsem2)
    osem = (osem0, osem1, osem2)

    wid = lax.axis_index("s") * _NC + lax.axis_index("c")
    p0 = wid * _P_W

    hold_cp = pltpu.async_copy(pos_hbm.at[pl.ds(p0, _P_W)], pos_v, hsem)

    for b in range(_BATCH):
        pltpu.sync_copy(x_hbm.at[pl.ds(b * _MAX_LEN + p0, _P_W)],
                        idx_v.at[pl.ds(b * _P_W, _P_W)])

    def fire_gather(c):
        b, q = divmod(c, _NQ)
        return pltpu.async_copy(
            table_hbm.at[idx_v.at[pl.ds(b * _P_W + q * _CH, _CH)]],
            rows[c % _NR], gsem[c % _NR])

    g_cp = [None] * _NCH
    o_cp = [None] * _NCH
    g_cp[0] = fire_gather(0)
    g_cp[1] = fire_gather(1)

    for c in range(_NCH):
        b, q = divmod(c, _NQ)
        rb = rows[c % _NR]
        g_cp[c].wait()
        if c == 0:
            hold_cp.wait()

        shift16 = jnp.full((_L,), 16, jnp.int32)
        mask_hi = jnp.full((_L,), _MASK_HI, jnp.int32)

        def row(i, carry, rb=rb, q=q):
            for g in range(_D // 32):
                w = pos_v[q * _CH + i, pl.ds(g * _L, _L)]
                lo = lax.bitcast_convert_type(lax.shift_left(w, shift16),
                                              jnp.float32)
                hi = lax.bitcast_convert_type(jnp.bitwise_and(w, mask_hi),
                                              jnp.float32)
                plsc.addupdate(rb.at[i, pl.ds(32 * g, _L)], lo)
                plsc.addupdate(rb.at[i, pl.ds(32 * g + _L, _L)], hi)
            return carry

        lax.fori_loop(0, _CH, row, 0)

        o_cp[c] = pltpu.async_copy(
            rb, out_hbm.at[pl.ds(b * _MAX_LEN + p0 + q * _CH, _CH)],
            osem[c % _NR])
        if c + 2 < _NCH:
            if c >= 1:
                o_cp[c - 1].wait()
            g_cp[c + 2] = fire_gather(c + 2)

    for c in range(_NCH - _NR, _NCH):
        o_cp[c].wait()


def kernel(x, start_token, end_token, table):
    batch, seq_len = x.shape
    out = _emb_kernel(x.reshape(-1), jnp.asarray(_POS_PACKED), table)
    return out.reshape(batch, seq_len, _D)


# speedup vs baseline: 1.1412x; 1.1412x over previous
import functools

import numpy as np
import jax
import jax.numpy as jnp
from jax import lax
from jax.experimental import pallas as pl
from jax.experimental.pallas import tpu as pltpu
from jax.experimental.pallas import tpu_sc as plsc

_BATCH = 4
_MAX_LEN = 2048
_D = 1024

_NC = 2
_NS = 16
_NW = _NC * _NS
_L = 16

_P_W = _MAX_LEN // _NW
_PER_W = _BATCH * _P_W
_CH = 8
_NQ = _P_W // _CH
_NG = 3
_NPB = 2


def _positional_encoding() -> np.ndarray:
    pos = np.arange(_MAX_LEN, dtype=np.float32)[:, None]
    i = np.arange(0, _D, 2, dtype=np.float32)
    div = np.exp(-np.log(10000.0) * i / _D)
    pe = np.zeros((_MAX_LEN, _D), dtype=np.float32)
    pe[:, 0::2] = np.sin(pos * div)
    pe[:, 1::2] = np.cos(pos * div)
    return pe


_POS = _positional_encoding()

_mesh = plsc.VectorSubcoreMesh(core_axis_name="c", subcore_axis_name="s")


@functools.partial(
    pl.kernel,
    mesh=_mesh,
    out_type=jax.ShapeDtypeStruct((_BATCH * _MAX_LEN, _D), jnp.float32),
    scratch_types=(
        [pltpu.VMEM((_PER_W,), jnp.int32)]
        + [pltpu.VMEM((_CH, _D), jnp.float32)] * (_NG * _BATCH + _NPB)
        + [pltpu.SemaphoreType.DMA] * (_NPB + 2 * _NG * _BATCH)
    ),
)
def _emb_kernel(x_hbm, pos_hbm, table_hbm, out_hbm, idx_v, *bufs_and_sems):
    rows = bufs_and_sems[:_NG * _BATCH]
    pbuf = bufs_and_sems[_NG * _BATCH:_NG * _BATCH + _NPB]
    sems = bufs_and_sems[_NG * _BATCH + _NPB:]
    psem = sems[:_NPB]
    gsem = sems[_NPB:_NPB + _NG * _BATCH]
    osem = sems[_NPB + _NG * _BATCH:]

    wid = lax.axis_index("s") * _NC + lax.axis_index("c")
    p0 = wid * _P_W

    for b in range(_BATCH):
        pltpu.sync_copy(x_hbm.at[pl.ds(b * _MAX_LEN + p0, _P_W)],
                        idx_v.at[pl.ds(b * _P_W, _P_W)])

    def fire_posload(q):
        return pltpu.async_copy(pos_hbm.at[pl.ds(p0 + q * _CH, _CH)],
                                pbuf[q % _NPB], psem[q % _NPB])

    def fire_gather(q, b):
        k = (q % _NG) * _BATCH + b
        return pltpu.async_copy(
            table_hbm.at[idx_v.at[pl.ds(b * _P_W + q * _CH, _CH)]],
            rows[k], gsem[k])

    def fire_out(q, b):
        k = (q % _NG) * _BATCH + b
        return pltpu.async_copy(
            rows[k], out_hbm.at[pl.ds(b * _MAX_LEN + p0 + q * _CH, _CH)],
            osem[k])

    g_cp = [[None] * _BATCH for _ in range(_NQ)]
    o_cp = [[None] * _BATCH for _ in range(_NQ)]
    p_cp = [None] * _NQ

    p_cp[0] = fire_posload(0)
    p_cp[1] = fire_posload(1)
    for q in range(2):
        for b in range(_BATCH):
            g_cp[q][b] = fire_gather(q, b)

    for q in range(_NQ):
        p_cp[q].wait()
        for b in range(_BATCH):
            g_cp[q][b].wait()

        pb = pbuf[q % _NPB]
        rbs = [rows[(q % _NG) * _BATCH + b] for b in range(_BATCH)]

        def row(i, carry, pb=pb, rbs=rbs):
            for j in range(_D // _L):
                sl = pl.ds(j * _L, _L)
                v = pb[i, sl]
                for rb in rbs:
                    plsc.addupdate(rb.at[i, sl], v)
            return carry

        lax.fori_loop(0, _CH, row, 0)

        if q + _NPB < _NQ:
            p_cp[q + _NPB] = fire_posload(q + _NPB)

        for b in range(_BATCH):
            o_cp[q][b] = fire_out(q, b)

        if q + 2 < _NQ:
            if q >= 1:
                for b in range(_BATCH):
                    o_cp[q - 1][b].wait()
            for b in range(_BATCH):
                g_cp[q + 2][b] = fire_gather(q + 2, b)

    for q in range(_NQ - 3, _NQ):
        for b in range(_BATCH):
            o_cp[q][b].wait()


def kernel(x, start_token, end_token, table):
    batch, seq_len = x.shape
    out = _emb_kernel(x.reshape(-1), jnp.asarray(_POS), table)
    return out.reshape(batch, seq_len, _D)


# speedup vs baseline: 1.1684x; 1.0239x over previous
import functools

import numpy as np
import jax
import jax.numpy as jnp
from jax import lax
from jax.experimental import pallas as pl
from jax.experimental.pallas import tpu as pltpu
from jax.experimental.pallas import tpu_sc as plsc

_BATCH = 4
_MAX_LEN = 2048
_D = 1024

_NC = 2
_NS = 16
_NW = _NC * _NS
_L = 16

_P_W = _MAX_LEN // _NW
_PER_W = _BATCH * _P_W
_CH = 8
_NQ = _P_W // _CH
_NG = 3
_NPB = 2


def _positional_encoding() -> np.ndarray:
    pos = np.arange(_MAX_LEN, dtype=np.float32)[:, None]
    i = np.arange(0, _D, 2, dtype=np.float32)
    div = np.exp(-np.log(10000.0) * i / _D)
    pe = np.zeros((_MAX_LEN, _D), dtype=np.float32)
    pe[:, 0::2] = np.sin(pos * div)
    pe[:, 1::2] = np.cos(pos * div)
    return pe


_POS = _positional_encoding()

_mesh = plsc.VectorSubcoreMesh(core_axis_name="c", subcore_axis_name="s")


@functools.partial(
    pl.kernel,
    mesh=_mesh,
    out_type=jax.ShapeDtypeStruct((_BATCH, _MAX_LEN, _D), jnp.float32),
    scratch_types=(
        [pltpu.VMEM((_PER_W,), jnp.int32)]
        + [pltpu.VMEM((_CH, _D), jnp.float32)] * (_NG * _BATCH + _NPB)
        + [pltpu.SemaphoreType.DMA] * (_NPB + 2 * _NG * _BATCH)
    ),
)
def _emb_kernel(x_hbm, pos_hbm, table_hbm, out_hbm, idx_v, *bufs_and_sems):
    rows = bufs_and_sems[:_NG * _BATCH]
    pbuf = bufs_and_sems[_NG * _BATCH:_NG * _BATCH + _NPB]
    sems = bufs_and_sems[_NG * _BATCH + _NPB:]
    psem = sems[:_NPB]
    gsem = sems[_NPB:_NPB + _NG * _BATCH]
    osem = sems[_NPB + _NG * _BATCH:]

    wid = lax.axis_index("s") * _NC + lax.axis_index("c")
    p0 = wid * _P_W

    for b in range(_BATCH):
        pltpu.sync_copy(x_hbm.at[b, pl.ds(p0, _P_W)],
                        idx_v.at[pl.ds(b * _P_W, _P_W)])

    def fire_posload(q):
        return pltpu.async_copy(pos_hbm.at[pl.ds(p0 + q * _CH, _CH)],
                                pbuf[q % _NPB], psem[q % _NPB])

    def fire_gather(q, b):
        k = (q % _NG) * _BATCH + b
        return pltpu.async_copy(
            table_hbm.at[idx_v.at[pl.ds(b * _P_W + q * _CH, _CH)]],
            rows[k], gsem[k])

    def fire_out(q, b):
        k = (q % _NG) * _BATCH + b
        return pltpu.async_copy(
            rows[k], out_hbm.at[b, pl.ds(p0 + q * _CH, _CH)],
            osem[k])

    g_cp = [[None] * _BATCH for _ in range(_NQ)]
    o_cp = [[None] * _BATCH for _ in range(_NQ)]
    p_cp = [None] * _NQ

    p_cp[0] = fire_posload(0)
    p_cp[1] = fire_posload(1)
    for q in range(2):
        for b in range(_BATCH):
            g_cp[q][b] = fire_gather(q, b)

    for q in range(_NQ):
        p_cp[q].wait()
        for b in range(_BATCH):
            g_cp[q][b].wait()

        pb = pbuf[q % _NPB]
        rbs = [rows[(q % _NG) * _BATCH + b] for b in range(_BATCH)]

        def row(i, carry, pb=pb, rbs=rbs):
            for j in range(_D // _L):
                sl = pl.ds(j * _L, _L)
                v = pb[i, sl]
                for rb in rbs:
                    plsc.addupdate(rb.at[i, sl], v)
            return carry

        lax.fori_loop(0, _CH, row, 0)

        if q + _NPB < _NQ:
            p_cp[q + _NPB] = fire_posload(q + _NPB)

        for b in range(_BATCH):
            o_cp[q][b] = fire_out(q, b)

        if q + 2 < _NQ:
            if q >= 1:
                for b in range(_BATCH):
                    o_cp[q - 1][b].wait()
            for b in range(_BATCH):
                g_cp[q + 2][b] = fire_gather(q + 2, b)

    for q in range(_NQ - 3, _NQ):
        for b in range(_BATCH):
            o_cp[q][b].wait()


def kernel(x, start_token, end_token, table):
    return _emb_kernel(x, jnp.asarray(_POS), table)
